# baseline (speedup 1.0000x reference)
import jax
import jax.numpy as jnp
from jax import lax
from jax.experimental import pallas as pl
from jax.experimental.pallas import tpu as pltpu
from jax.experimental.pallas import tpu_sc as plsc

BATCH = 16384
N_FIELDS = 26
LINEAR_SIZE = 13
LANES = 16
CHUNK = 128
N_Q = 4


def _sc_linear(table_hbm, idx_hbm, dense_hbm, w_hbm, b_hbm, out_hbm,
               iq0, iq1, iq2, iq3, vq0, vq1, vq2, vq3,
               dense_v, w_v, b_v, out_v, isems, gsems, dsem):
    info = plsc.get_sparse_core_info()
    nc, ns = info.num_cores, info.num_subcores
    nw = nc * ns
    rows = BATCH // nw
    n_groups = rows // LANES

    wid = lax.axis_index("s") * nc + lax.axis_index("c")
    base = wid * rows
    iqs = (iq0, iq1, iq2, iq3)
    vqs = (vq0, vq1, vq2, vq3)

    for k in range(N_Q):
        pltpu.make_async_copy(
            idx_hbm.at[:, pl.ds(base + k * CHUNK, CHUNK)], iqs[k],
            isems.at[k],
        ).start()
    dense_cp = pltpu.make_async_copy(
        dense_hbm.at[:, pl.ds(base, rows)], dense_v, dsem)
    dense_cp.start()
    pltpu.sync_copy(w_hbm.at[0], w_v.at[pl.ds(0, LINEAR_SIZE)])
    pltpu.sync_copy(b_hbm, b_v.at[pl.ds(0, 1)])

    for k in range(N_Q):
      with jax.named_scope(f"fire_q{k}"):
        pltpu.make_async_copy(
            idx_hbm.at[:, pl.ds(base + k * CHUNK, CHUNK)], iqs[k],
            isems.at[k],
        ).wait()
        iq = iqs[k]
        vq = vqs[k]

        for f in range(N_FIELDS):
            pltpu.make_async_copy(
                table_hbm.at[0].at[iq.at[f]], vq.at[f], gsems.at[k],
            ).start()

    wvec = w_v[...]
    w = [wvec[j] for j in range(LINEAR_SIZE)]
    b = b_v[...][0]
    with jax.named_scope("dense_wait"):
        dense_cp.wait()

    def seed(g, carry):
        goff = pl.multiple_of(g * LANES, LANES)
        acc = jnp.full((LANES,), b, dtype=jnp.float32)
        for j in range(LINEAR_SIZE):
            acc = acc + dense_v[j, pl.ds(goff, LANES)] * w[j]
        out_v[pl.ds(goff, LANES)] = acc
        return carry

    with jax.named_scope("seed"):
        lax.fori_loop(0, n_groups, seed, 0)

    for k in range(N_Q):
      with jax.named_scope(f"drain_q{k}"):
        iq = iqs[k]
        vq = vqs[k]

        def acc_f(f, carry, iq=iq, vq=vq, k=k):
            pltpu.make_async_copy(
                table_hbm.at[0].at[iq.at[f]], vq.at[f], gsems.at[k],
            ).wait()
            for gq in range(CHUNK // LANES):
                off = pl.multiple_of(gq * LANES, LANES)
                plsc.addupdate(
                    out_v.at[pl.ds(k * CHUNK + off, LANES)],
                    vq[f, pl.ds(off, LANES)])
            return carry

        lax.fori_loop(0, N_FIELDS, acc_f, 0)

    pltpu.sync_copy(out_v, out_hbm.at[pl.ds(base, rows)])


def kernel(dense_input, sparse_input, W_dense, b_dense, W_sparse):
    info = plsc.get_sparse_core_info()
    nw = info.num_cores * info.num_subcores
    rows = BATCH // nw

    idx_t = sparse_input.astype(jnp.int32).T
    dense_t = dense_input.T
    table = W_sparse.T
    w_t = W_dense.T

    mesh = plsc.VectorSubcoreMesh(core_axis_name="c", subcore_axis_name="s")
    run = pl.kernel(
        _sc_linear,
        mesh=mesh,
        out_type=jax.ShapeDtypeStruct((BATCH,), jnp.float32),
        scratch_types=(
            [pltpu.VMEM((N_FIELDS, CHUNK), jnp.int32)] * N_Q
            + [pltpu.VMEM((N_FIELDS, CHUNK), jnp.float32)] * N_Q
            + [
                pltpu.VMEM((LINEAR_SIZE, rows), jnp.float32),
                pltpu.VMEM((LANES,), jnp.float32),
                pltpu.VMEM((LANES,), jnp.float32),
                pltpu.VMEM((rows,), jnp.float32),
                pltpu.SemaphoreType.DMA((N_Q,)),
                pltpu.SemaphoreType.DMA((N_Q,)),
                pltpu.SemaphoreType.DMA,
            ]
        ),
    )
    out = run(table, idx_t, dense_t, w_t, b_dense)
    return out.reshape(BATCH, 1)


# speedup vs baseline: 1.0392x; 1.0392x over previous
import jax
import jax.numpy as jnp
from jax import lax
from jax.experimental import pallas as pl
from jax.experimental.pallas import tpu as pltpu
from jax.experimental.pallas import tpu_sc as plsc

BATCH = 16384
N_FIELDS = 26
LINEAR_SIZE = 13
LANES = 16
CHUNK = 128
HALVES = ((0, 16), (16, 10))


def _sc_linear(table_hbm, idx_hbm, dense_hbm, w_hbm, b_hbm, out_hbm,
               idx_v, vals_v, dense_v, w_v, b_v, out_v, isems, gsems, dsem):
    info = plsc.get_sparse_core_info()
    nc, ns = info.num_cores, info.num_subcores
    nw = nc * ns
    rows = BATCH // nw
    n_groups = rows // LANES
    q_per_f = rows // CHUNK

    wid = lax.axis_index("s") * nc + lax.axis_index("c")
    base = wid * rows

    for h, (lo, n) in enumerate(HALVES):
        pltpu.make_async_copy(
            idx_hbm.at[pl.ds(lo, n), pl.ds(base, rows)],
            idx_v.at[pl.ds(lo, n)],
            isems.at[h],
        ).start()
    dense_cp = pltpu.make_async_copy(
        dense_hbm.at[:, pl.ds(base, rows)], dense_v, dsem)
    dense_cp.start()
    pltpu.sync_copy(w_hbm.at[0], w_v.at[pl.ds(0, LINEAR_SIZE)])
    pltpu.sync_copy(b_hbm, b_v.at[pl.ds(0, 1)])

    for h, (lo, n) in enumerate(HALVES):
        pltpu.make_async_copy(
            idx_hbm.at[pl.ds(lo, n), pl.ds(base, rows)],
            idx_v.at[pl.ds(lo, n)],
            isems.at[h],
        ).wait()
        for f in range(lo, lo + n):
            for q in range(q_per_f):
                pltpu.make_async_copy(
                    table_hbm.at[0].at[idx_v.at[f].at[pl.ds(q * CHUNK, CHUNK)]],
                    vals_v.at[f].at[pl.ds(q * CHUNK, CHUNK)],
                    gsems.at[f],
                ).start()

    wvec = w_v[...]
    w = [wvec[j] for j in range(LINEAR_SIZE)]
    b = b_v[...][0]
    dense_cp.wait()

    def seed(g, carry):
        goff = pl.multiple_of(g * LANES, LANES)
        acc = jnp.full((LANES,), b, dtype=jnp.float32)
        for j in range(LINEAR_SIZE):
            acc = acc + dense_v[j, pl.ds(goff, LANES)] * w[j]
        out_v[pl.ds(goff, LANES)] = acc
        return carry

    lax.fori_loop(0, n_groups, seed, 0)

    def acc_f(f, carry):
        pltpu.make_async_copy(
            table_hbm.at[0].at[idx_v.at[f]], vals_v.at[f], gsems.at[f],
        ).wait()

        def acc_g(g, c2):
            goff = pl.multiple_of(g * LANES, LANES)
            plsc.addupdate(out_v.at[pl.ds(goff, LANES)],
                           vals_v[f, pl.ds(goff, LANES)])
            return c2

        lax.fori_loop(0, n_groups, acc_g, 0)
        return carry

    lax.fori_loop(0, N_FIELDS, acc_f, 0)

    pltpu.sync_copy(out_v, out_hbm.at[pl.ds(base, rows)])


def kernel(dense_input, sparse_input, W_dense, b_dense, W_sparse):
    info = plsc.get_sparse_core_info()
    nw = info.num_cores * info.num_subcores
    rows = BATCH // nw

    idx_t = sparse_input.astype(jnp.int32).T
    dense_t = dense_input.T
    table = W_sparse.T
    w_t = W_dense.T

    mesh = plsc.VectorSubcoreMesh(core_axis_name="c", subcore_axis_name="s")
    run = pl.kernel(
        _sc_linear,
        mesh=mesh,
        out_type=jax.ShapeDtypeStruct((BATCH,), jnp.float32),
        scratch_types=[
            pltpu.VMEM((N_FIELDS, rows), jnp.int32),
            pltpu.VMEM((N_FIELDS, rows), jnp.float32),
            pltpu.VMEM((LINEAR_SIZE, rows), jnp.float32),
            pltpu.VMEM((LANES,), jnp.float32),
            pltpu.VMEM((LANES,), jnp.float32),
            pltpu.VMEM((rows,), jnp.float32),
            pltpu.SemaphoreType.DMA((2,)),
            pltpu.SemaphoreType.DMA((N_FIELDS,)),
            pltpu.SemaphoreType.DMA,
        ],
    )
    out = run(table, idx_t, dense_t, w_t, b_dense)
    return out.reshape(BATCH, 1)
